# baseline (speedup 1.0000x reference)
import functools

import jax
import jax.numpy as jnp
from jax import lax
from jax.experimental import pallas as pl
from jax.experimental.pallas import tpu as pltpu
from jax.experimental.pallas import tpu_sc as plsc

B = 4
S = 4096
D = 1024
NC = 2
NS = 16
NW = NC * NS
CHUNK = S // NW
ROWS = 16
NBLK = CHUNK // ROWS
NVEC = D // 16


def _body(x_hbm, scale_hbm, pos_hbm, tse_hbm, out_hbm,
          xbuf, posbuf, trow, idxv, sem):
    wid = lax.axis_index("s") * NC + lax.axis_index("c")
    base = wid * CHUNK

    pltpu.sync_copy(scale_hbm, idxv)
    pltpu.async_copy(tse_hbm.at[idxv], trow, sem).wait()

    def block(blk, carry):
        s0 = base + blk * ROWS
        pltpu.sync_copy(pos_hbm.at[pl.ds(s0, ROWS)], posbuf)
        for b in range(B):
            pltpu.sync_copy(x_hbm.at[b, pl.ds(s0, ROWS)], xbuf.at[b])

        def row(s, carry2):
            for j in range(NVEC):
                dj = pl.ds(j * 16, 16)
                pv = posbuf[s, dj] + trow[0, dj]
                for b in range(B):
                    xbuf[b, s, dj] = xbuf[b, s, dj] + pv
            return carry2

        lax.fori_loop(0, ROWS, row, 0)

        for b in range(B):
            pltpu.sync_copy(xbuf.at[b], out_hbm.at[b, pl.ds(s0, ROWS)])
        return carry

    lax.fori_loop(0, NBLK, block, 0)


@jax.jit
def _run(x, scale_arr, pos_embedding, tse):
    mesh = plsc.VectorSubcoreMesh(core_axis_name="c", subcore_axis_name="s")
    kfn = pl.kernel(
        _body,
        out_type=jax.ShapeDtypeStruct((B, S, D), jnp.float32),
        mesh=mesh,
        scratch_types=[
            pltpu.VMEM((B, ROWS, D), jnp.float32),
            pltpu.VMEM((ROWS, D), jnp.float32),
            pltpu.VMEM((1, D), jnp.float32),
            pltpu.VMEM((1,), jnp.int32),
            pltpu.SemaphoreType.DMA,
        ],
    )
    return kfn(x, scale_arr, pos_embedding, tse)


def kernel(x, time_scale, pos_embedding, time_scale_embedding):
    ts = jnp.asarray(time_scale).astype(jnp.float32)
    scale_idx = jnp.minimum(jnp.log2(ts).astype(jnp.int32), 9)
    scale_arr = scale_idx.reshape((1,)).astype(jnp.int32)
    return _run(x, scale_arr, pos_embedding, time_scale_embedding)


# speedup vs baseline: 1.6425x; 1.6425x over previous
import jax
import jax.numpy as jnp
from jax import lax
from jax.experimental import pallas as pl
from jax.experimental.pallas import tpu as pltpu
from jax.experimental.pallas import tpu_sc as plsc

B = 4
S = 4096
D = 1024
NC = 2
NS = 16
NW = NC * NS
CHUNK = S // NW
ROWS = 8
NBLK = CHUNK // ROWS
NVEC = D // 16


def _body(x_hbm, scale_hbm, pos_hbm, tse_hbm, out_hbm,
          xbuf, posbuf, trow, idxv, isem0, isem1, osem0, osem1, gsem):
    wid = lax.axis_index("s") * NC + lax.axis_index("c")
    base = wid * CHUNK
    isem = (isem0, isem1)
    osem = (osem0, osem1)

    pltpu.sync_copy(scale_hbm, idxv)
    pltpu.async_copy(tse_hbm.at[idxv], trow, gsem).wait()

    def in_descs(buf, blk):
        s0 = base + blk * ROWS
        return (
            pltpu.make_async_copy(pos_hbm.at[pl.ds(s0, ROWS)],
                                  posbuf.at[buf], isem[buf]),
            pltpu.make_async_copy(x_hbm.at[:, pl.ds(s0, ROWS)],
                                  xbuf.at[buf], isem[buf]),
        )

    def out_desc(buf, blk):
        s0 = base + blk * ROWS
        return pltpu.make_async_copy(xbuf.at[buf],
                                     out_hbm.at[:, pl.ds(s0, ROWS)],
                                     osem[buf])

    def fire_in(buf, blk):
        for d in in_descs(buf, blk):
            d.start()

    def drain_in(buf, blk):
        for d in in_descs(buf, blk):
            d.wait()

    def compute(buf):
        def row(s, carry):
            for j in range(NVEC):
                dj = pl.ds(j * 16, 16)
                pv = posbuf[buf, s, dj] + trow[0, dj]
                for b in range(B):
                    xbuf[buf, b, s, dj] = xbuf[buf, b, s, dj] + pv
            return carry
        lax.fori_loop(0, ROWS, row, 0)

    fire_in(0, 0)

    def step(it, carry):
        for phase in range(2):
            blk = it * 2 + phase
            cur, nxt = phase, 1 - phase

            @pl.when(blk >= 1)
            def _():
                out_desc(nxt, blk - 1).wait()

            @pl.when(blk + 1 < NBLK)
            def _():
                fire_in(nxt, blk + 1)

            drain_in(cur, blk)
            compute(cur)
            out_desc(cur, blk).start()
        return carry

    lax.fori_loop(0, NBLK // 2, step, 0)
    out_desc((NBLK - 1) % 2, NBLK - 1).wait()


@jax.jit
def _run(x, scale_arr, pos_embedding, tse):
    mesh = plsc.VectorSubcoreMesh(core_axis_name="c", subcore_axis_name="s")
    kfn = pl.kernel(
        _body,
        out_type=jax.ShapeDtypeStruct((B, S, D), jnp.float32),
        mesh=mesh,
        scratch_types=[
            pltpu.VMEM((2, B, ROWS, D), jnp.float32),
            pltpu.VMEM((2, ROWS, D), jnp.float32),
            pltpu.VMEM((1, D), jnp.float32),
            pltpu.VMEM((1,), jnp.int32),
            pltpu.SemaphoreType.DMA,
            pltpu.SemaphoreType.DMA,
            pltpu.SemaphoreType.DMA,
            pltpu.SemaphoreType.DMA,
            pltpu.SemaphoreType.DMA,
        ],
    )
    return kfn(x, scale_arr, pos_embedding, tse)


def kernel(x, time_scale, pos_embedding, time_scale_embedding):
    ts = jnp.asarray(time_scale).astype(jnp.float32)
    scale_idx = jnp.minimum(jnp.log2(ts).astype(jnp.int32), 9)
    scale_arr = scale_idx.reshape((1,)).astype(jnp.int32)
    return _run(x, scale_arr, pos_embedding, time_scale_embedding)


# speedup vs baseline: 2.6682x; 1.6245x over previous
import jax
import jax.numpy as jnp
from jax import lax
from jax.experimental import pallas as pl
from jax.experimental.pallas import tpu as pltpu
from jax.experimental.pallas import tpu_sc as plsc

B = 4
S = 4096
D = 1024
NC = 2
NS = 16
NW = NC * NS
CHUNK = S // NW
ROWS = 8
NBLK = CHUNK // ROWS
NVEC = D // 16


def _body(x_hbm, scale_hbm, pos_hbm, tse_hbm, out_hbm,
          xbuf, posbuf, trow, idxv, isem0, isem1, osem0, osem1, gsem):
    wid = lax.axis_index("s") * NC + lax.axis_index("c")
    base = wid * CHUNK
    isem = (isem0, isem1)
    osem = (osem0, osem1)

    pltpu.sync_copy(scale_hbm, idxv)
    pltpu.async_copy(tse_hbm.at[idxv], trow, gsem).wait()

    def in_descs(buf, blk):
        s0 = base + blk * ROWS
        return (
            pltpu.make_async_copy(pos_hbm.at[pl.ds(s0, ROWS)],
                                  posbuf.at[buf], isem[buf]),
            pltpu.make_async_copy(x_hbm.at[:, pl.ds(s0, ROWS)],
                                  xbuf.at[buf], isem[buf]),
        )

    def out_desc(buf, blk):
        s0 = base + blk * ROWS
        return pltpu.make_async_copy(xbuf.at[buf],
                                     out_hbm.at[:, pl.ds(s0, ROWS)],
                                     osem[buf])

    def fire_in(buf, blk):
        for d in in_descs(buf, blk):
            d.start()

    def drain_in(buf, blk):
        for d in in_descs(buf, blk):
            d.wait()

    def compute(buf):
        U = 4

        def row(s, carry):
            for j0 in range(0, NVEC, U):
                pvs = []
                for u in range(U):
                    dj = pl.ds((j0 + u) * 16, 16)
                    pvs.append(posbuf[buf, s, dj] + trow[0, dj])
                for u in range(U):
                    dj = pl.ds((j0 + u) * 16, 16)
                    for b in range(B):
                        plsc.addupdate(xbuf.at[buf, b, s, dj], pvs[u])
            return carry
        lax.fori_loop(0, ROWS, row, 0)

    fire_in(0, 0)

    def step(it, carry):
        for phase in range(2):
            blk = it * 2 + phase
            cur, nxt = phase, 1 - phase

            @pl.when(blk >= 1)
            def _():
                out_desc(nxt, blk - 1).wait()

            @pl.when(blk + 1 < NBLK)
            def _():
                fire_in(nxt, blk + 1)

            drain_in(cur, blk)
            compute(cur)
            out_desc(cur, blk).start()
        return carry

    lax.fori_loop(0, NBLK // 2, step, 0)
    out_desc((NBLK - 1) % 2, NBLK - 1).wait()


@jax.jit
def _run(x, scale_arr, pos_embedding, tse):
    mesh = plsc.VectorSubcoreMesh(core_axis_name="c", subcore_axis_name="s")
    kfn = pl.kernel(
        _body,
        out_type=jax.ShapeDtypeStruct((B, S, D), jnp.float32),
        mesh=mesh,
        scratch_types=[
            pltpu.VMEM((2, B, ROWS, D), jnp.float32),
            pltpu.VMEM((2, ROWS, D), jnp.float32),
            pltpu.VMEM((1, D), jnp.float32),
            pltpu.VMEM((1,), jnp.int32),
            pltpu.SemaphoreType.DMA,
            pltpu.SemaphoreType.DMA,
            pltpu.SemaphoreType.DMA,
            pltpu.SemaphoreType.DMA,
            pltpu.SemaphoreType.DMA,
        ],
    )
    return kfn(x, scale_arr, pos_embedding, tse)


def kernel(x, time_scale, pos_embedding, time_scale_embedding):
    ts = jnp.asarray(time_scale).astype(jnp.float32)
    scale_idx = jnp.minimum(jnp.log2(ts).astype(jnp.int32), 9)
    scale_arr = scale_idx.reshape((1,)).astype(jnp.int32)
    return _run(x, scale_arr, pos_embedding, time_scale_embedding)


# speedup vs baseline: 2.6932x; 1.0093x over previous
import jax
import jax.numpy as jnp
from jax import lax
from jax.experimental import pallas as pl
from jax.experimental.pallas import tpu as pltpu
from jax.experimental.pallas import tpu_sc as plsc

B = 4
S = 4096
D = 1024
NC = 2
NS = 16
NW = NC * NS
CHUNK = S // NW
ROWS = 8
NBLK = CHUNK // ROWS
NVEC = D // 16


def _body(x_hbm, scale_hbm, pos_hbm, tse_hbm, out_hbm,
          xbuf, posbuf, trow, idxv, isem0, isem1, osem0, osem1, gsem):
    wid = lax.axis_index("s") * NC + lax.axis_index("c")
    base = wid * CHUNK
    isem = (isem0, isem1)
    osem = (osem0, osem1)

    pltpu.sync_copy(scale_hbm, idxv)
    pltpu.async_copy(tse_hbm.at[idxv], trow, gsem).wait()

    def in_descs(buf, blk):
        s0 = base + blk * ROWS
        return (
            pltpu.make_async_copy(pos_hbm.at[pl.ds(s0, ROWS)],
                                  posbuf.at[buf], isem[buf]),
            pltpu.make_async_copy(x_hbm.at[:, pl.ds(s0, ROWS)],
                                  xbuf.at[buf], isem[buf]),
        )

    def out_desc(buf, blk):
        s0 = base + blk * ROWS
        return pltpu.make_async_copy(xbuf.at[buf],
                                     out_hbm.at[:, pl.ds(s0, ROWS)],
                                     osem[buf])

    def fire_in(buf, blk):
        for d in in_descs(buf, blk):
            d.start()

    def drain_in(buf, blk):
        for d in in_descs(buf, blk):
            d.wait()

    def compute(buf):
        def jblock(j, carry):
            dj = pl.ds(pl.multiple_of(j * 16, 16), 16)
            tv = trow[0, dj]
            pvs = [posbuf[buf, s, dj] + tv for s in range(ROWS)]
            for s in range(ROWS):
                for b in range(B):
                    plsc.addupdate(xbuf.at[buf, b, s, dj], pvs[s])
            return carry
        lax.fori_loop(0, NVEC, jblock, 0)

    fire_in(0, 0)

    def step(it, carry):
        for phase in range(2):
            blk = it * 2 + phase
            cur, nxt = phase, 1 - phase

            @pl.when(blk >= 1)
            def _():
                out_desc(nxt, blk - 1).wait()

            @pl.when(blk + 1 < NBLK)
            def _():
                fire_in(nxt, blk + 1)

            drain_in(cur, blk)
            compute(cur)
            out_desc(cur, blk).start()
        return carry

    lax.fori_loop(0, NBLK // 2, step, 0)
    out_desc((NBLK - 1) % 2, NBLK - 1).wait()


@jax.jit
def _run(x, scale_arr, pos_embedding, tse):
    mesh = plsc.VectorSubcoreMesh(core_axis_name="c", subcore_axis_name="s")
    kfn = pl.kernel(
        _body,
        out_type=jax.ShapeDtypeStruct((B, S, D), jnp.float32),
        mesh=mesh,
        scratch_types=[
            pltpu.VMEM((2, B, ROWS, D), jnp.float32),
            pltpu.VMEM((2, ROWS, D), jnp.float32),
            pltpu.VMEM((1, D), jnp.float32),
            pltpu.VMEM((1,), jnp.int32),
            pltpu.SemaphoreType.DMA,
            pltpu.SemaphoreType.DMA,
            pltpu.SemaphoreType.DMA,
            pltpu.SemaphoreType.DMA,
            pltpu.SemaphoreType.DMA,
        ],
    )
    return kfn(x, scale_arr, pos_embedding, tse)


def kernel(x, time_scale, pos_embedding, time_scale_embedding):
    ts = jnp.asarray(time_scale).astype(jnp.float32)
    scale_idx = jnp.minimum(jnp.log2(ts).astype(jnp.int32), 9)
    scale_arr = scale_idx.reshape((1,)).astype(jnp.int32)
    return _run(x, scale_arr, pos_embedding, time_scale_embedding)
